# Initial kernel scaffold; baseline (speedup 1.0000x reference)
import jax
import jax.numpy as jnp
from jax.experimental import pallas as pl


def kernel(adjacency_indices, adjacency_values, input_features, W, bias):
    raise NotImplementedError("write your pallas kernel here")


# speedup vs baseline: 5.4240x; 5.4240x over previous
import functools

import jax
import jax.numpy as jnp
from jax import lax
from jax.experimental import pallas as pl
from jax.experimental.pallas import tpu as pltpu
from jax.experimental.pallas import tpu_sc as plsc

N_NODES = 10000
D = 128
N_EDGES = 320000

NC = 2
NS = 16
NW = NC * NS
LANES = 16

EDGES_PER_WORKER = N_EDGES // NW
CHUNK = 128
FULL_CHUNKS = EDGES_PER_WORKER // CHUNK
TAIL = EDGES_PER_WORKER - FULL_CHUNKS * CHUNK
ROWS_PER_TILE = 624
ZCHUNK = 208
EXTRA_BASE = ROWS_PER_TILE * NS


def _scale_rows(rows_ref, val_ref, n_edges):
    @pl.loop(0, n_edges // LANES)
    def _(g):
        vv = val_ref[pl.ds(g * LANES, LANES)]
        for l in range(LANES):
            v = vv[l]
            e = g * LANES + l
            for c in range(D // LANES):
                sl = pl.ds(c * LANES, LANES)
                rows_ref[e, sl] = rows_ref[e, sl] * v


def _sc_body(src_h, dst_h, val_h, x_h, out_h,
             acc, zbuf, srcv, dstv, valv, rows,
             srct, dstt, valt, rowst, gsem):
    c = lax.axis_index("c")
    s = lax.axis_index("s")
    wid = c * NS + s
    ebase = wid * EDGES_PER_WORKER

    @pl.loop(0, ZCHUNK)
    def _(i):
        for cv in range(D // LANES):
            zbuf[i, pl.ds(cv * LANES, LANES)] = jnp.zeros((LANES,), jnp.float32)

    @pl.loop(0, ROWS_PER_TILE // ZCHUNK)
    def _(k):
        pltpu.sync_copy(zbuf, acc.at[pl.ds(s * ROWS_PER_TILE + k * ZCHUNK, ZCHUNK)])

    @pl.when(s == NS - 1)
    def _():
        pltpu.sync_copy(zbuf.at[pl.ds(0, N_NODES - EXTRA_BASE)],
                        acc.at[pl.ds(EXTRA_BASE, N_NODES - EXTRA_BASE)])

    plsc.subcore_barrier()

    @pl.loop(0, FULL_CHUNKS)
    def _(j):
        base = ebase + j * CHUNK
        pltpu.sync_copy(src_h.at[pl.ds(base, CHUNK)], srcv)
        pltpu.sync_copy(dst_h.at[pl.ds(base, CHUNK)], dstv.at[0])
        pltpu.sync_copy(val_h.at[pl.ds(base, CHUNK)], valv)
        pltpu.async_copy(x_h.at[srcv], rows, gsem).wait()
        _scale_rows(rows, valv, CHUNK)
        pltpu.sync_copy(rows, acc.at[dstv.at[0]], add=True)

    tbase = ebase + FULL_CHUNKS * CHUNK
    pltpu.sync_copy(src_h.at[pl.ds(tbase, TAIL)], srct)
    pltpu.sync_copy(dst_h.at[pl.ds(tbase, TAIL)], dstt.at[0])
    pltpu.sync_copy(val_h.at[pl.ds(tbase, TAIL)], valt)
    pltpu.async_copy(x_h.at[srct], rowst, gsem).wait()
    _scale_rows(rowst, valt, TAIL)
    pltpu.sync_copy(rowst, acc.at[dstt.at[0]], add=True)

    plsc.subcore_barrier()
    rb = s * ROWS_PER_TILE
    pltpu.sync_copy(acc.at[pl.ds(rb, ROWS_PER_TILE)],
                    out_h.at[c, pl.ds(rb, ROWS_PER_TILE)])

    @pl.when(s == NS - 1)
    def _():
        pltpu.sync_copy(acc.at[pl.ds(EXTRA_BASE, N_NODES - EXTRA_BASE)],
                        out_h.at[c, pl.ds(EXTRA_BASE, N_NODES - EXTRA_BASE)])


_sc_scatter = pl.kernel(
    _sc_body,
    out_type=jax.ShapeDtypeStruct((NC, N_NODES, D), jnp.float32),
    mesh=plsc.VectorSubcoreMesh(
        core_axis_name="c", subcore_axis_name="s",
        num_cores=NC, num_subcores=NS),
    scratch_types=[
        pltpu.VMEM_SHARED((N_NODES, D), jnp.float32),
        pltpu.VMEM((ZCHUNK, D), jnp.float32),
        pltpu.VMEM((CHUNK,), jnp.int32),
        pltpu.VMEM((1, CHUNK), jnp.int32),
        pltpu.VMEM((CHUNK,), jnp.float32),
        pltpu.VMEM((CHUNK, D), jnp.float32),
        pltpu.VMEM((TAIL,), jnp.int32),
        pltpu.VMEM((1, TAIL), jnp.int32),
        pltpu.VMEM((TAIL,), jnp.float32),
        pltpu.VMEM((TAIL, D), jnp.float32),
        pltpu.SemaphoreType.DMA,
    ],
)


BR = 400


def _mm_body(p_ref, w_ref, b_ref, o_ref):
    z = p_ref[0] + p_ref[1]
    o_ref[...] = (
        jnp.dot(z, w_ref[...], preferred_element_type=jnp.float32) + b_ref[...]
    )


_tc_matmul = pl.pallas_call(
    _mm_body,
    grid=(N_NODES // BR,),
    in_specs=[
        pl.BlockSpec((NC, BR, D), lambda i: (0, i, 0)),
        pl.BlockSpec((D, D), lambda i: (0, 0)),
        pl.BlockSpec((1, D), lambda i: (0, 0)),
    ],
    out_specs=pl.BlockSpec((BR, D), lambda i: (i, 0)),
    out_shape=jax.ShapeDtypeStruct((N_NODES, D), jnp.float32),
)


@jax.jit
def kernel(adjacency_indices, adjacency_values, input_features, W, bias):
    dst = adjacency_indices[0]
    src = adjacency_indices[1]
    partials = _sc_scatter(src, dst, adjacency_values, input_features)
    return _tc_matmul(partials, W, bias.reshape(1, D))
